# baseline (speedup 1.0000x reference)
import functools

import numpy as np
import jax
import jax.numpy as jnp
from jax import lax
from jax.experimental import pallas as pl
from jax.experimental.pallas import tpu as pltpu
from jax.experimental.pallas import tpu_sc as plsc

_VOCAB = 1000000
_D = 32
_B = 4096
_L = 50
_NC = 2
_NS = 16
_NW = _NC * _NS
_BW = _B // _NW

_CH = 512
_NFULL = _VOCAB // _CH // _NW * _NW * (_VOCAB // _CH // _NW)
_CHUNKS = 1952
_PER_W = _CHUNKS // _NW
_TAIL = _VOCAB - _CHUNKS * _CH
_TAIL_OFF = _CHUNKS * _CH


def _pe_flat() -> np.ndarray:
    pos = np.arange(_L, dtype=np.float32)[:, None]
    div = np.exp(np.arange(0, _D, 2, dtype=np.float32) * (-np.log(10000.0) / _D))
    pe = np.zeros((_L, _D), np.float32)
    pe[:, 0::2] = np.sin(pos * div)
    pe[:, 1::2] = np.cos(pos * div)
    return np.repeat(pe.reshape(-1), 16)


def _make_mesh():
    return plsc.VectorSubcoreMesh(core_axis_name="c", subcore_axis_name="s")


def _relayout(tT):
    @functools.partial(
        pl.kernel,
        mesh=_make_mesh(),
        compiler_params=pltpu.CompilerParams(use_tc_tiling_on_sc=False, needs_layout_passes=False),
        out_type=jax.ShapeDtypeStruct((_VOCAB, _D), jnp.float32),
        scratch_types=(
            [pltpu.VMEM((_D, _CH), jnp.float32) for _ in range(2)]
            + [pltpu.VMEM((_CH, _D), jnp.float32) for _ in range(2)]
            + [pltpu.VMEM((_D, _TAIL), jnp.float32),
               pltpu.VMEM((_TAIL, _D), jnp.float32)]
            + [pltpu.SemaphoreType.DMA for _ in range(5)]
        ),
    )
    def run(tT_hbm, out_hbm, in0, in1, tr0, tr1, tin, ttr,
            gs0, gs1, ss0, ss1, tsem):
        wid = lax.axis_index("s") * _NC + lax.axis_index("c")
        ins = (in0, in1)
        trs = (tr0, tr1)
        gsems = (gs0, gs1)
        ssems = (ss0, ss1)
        d_lo = lax.iota(jnp.int32, 16)
        d_hi = d_lo + 16

        def col_of(i):
            return (wid + i * _NW) * _CH

        for b in range(2):
            pltpu.async_copy(tT_hbm.at[:, pl.ds(col_of(b), _CH)],
                             ins[b], gsems[b])

        def transpose_chunk(src, dst):
            def body(v8, carry):
                for u in range(8):
                    v = v8 * 8 + u
                    cols = jnp.full((16,), v, jnp.int32)
                    lo = plsc.load_gather(src, (d_lo, cols))
                    hi = plsc.load_gather(src, (d_hi, cols))
                    dst[v, pl.ds(0, 16)] = lo
                    dst[v, pl.ds(16, 16)] = hi
                return carry
            lax.fori_loop(0, _CH // 8, body, 0)

        def chunk_loop(i, carry):
            for b in range(2):
                j = i * 2 + b

                @pl.when(j < _PER_W)
                def _do():
                    pltpu.make_async_copy(
                        tT_hbm.at[:, pl.ds(col_of(j), _CH)], ins[b],
                        gsems[b]).wait()
                    @pl.when(j >= 2)
                    def _ws():
                        pltpu.make_async_copy(
                            trs[b], out_hbm.at[pl.ds(col_of(j - 2), _CH)],
                            ssems[b]).wait()

                    transpose_chunk(ins[b], trs[b])
                    pltpu.async_copy(trs[b],
                                     out_hbm.at[pl.ds(col_of(j), _CH)],
                                     ssems[b])
                    @pl.when(j + 2 < _PER_W)
                    def _pf():
                        pltpu.async_copy(tT_hbm.at[:, pl.ds(col_of(j + 2), _CH)],
                                         ins[b], gsems[b])
            return carry

        lax.fori_loop(0, (_PER_W + 1) // 2, chunk_loop, 0)

        for b in range(2):
            last = _PER_W - 1 - b
            pltpu.make_async_copy(trs[b],
                                  out_hbm.at[pl.ds(col_of(last), _CH)],
                                  ssems[b]).wait()

        @pl.when(wid == 0)
        def _tail():
            pltpu.async_copy(tT_hbm.at[:, pl.ds(_TAIL_OFF, _TAIL)], tin, tsem)
            pltpu.make_async_copy(tT_hbm.at[:, pl.ds(_TAIL_OFF, _TAIL)], tin,
                                  tsem).wait()

            def tbody(v8, carry):
                for u in range(8):
                    v = v8 * 8 + u
                    cols = jnp.full((16,), v, jnp.int32)
                    lo = plsc.load_gather(tin, (d_lo, cols))
                    hi = plsc.load_gather(tin, (d_hi, cols))
                    ttr[v, pl.ds(0, 16)] = lo
                    ttr[v, pl.ds(16, 16)] = hi
                return carry
            lax.fori_loop(0, _TAIL // 8, tbody, 0)
            pltpu.async_copy(ttr, out_hbm.at[pl.ds(_TAIL_OFF, _TAIL)], tsem)
            pltpu.make_async_copy(ttr, out_hbm.at[pl.ds(_TAIL_OFF, _TAIL)],
                                  tsem).wait()

    return run(tT)


def _gather_pe(table_rm, idxT, peb):
    @functools.partial(
        pl.kernel,
        mesh=_make_mesh(),
        compiler_params=pltpu.CompilerParams(use_tc_tiling_on_sc=False, needs_layout_passes=False),
        out_type=jax.ShapeDtypeStruct((_L, _D, _B), jnp.float32),
        scratch_types=(
            [pltpu.VMEM((_L, _BW), jnp.int32),
             pltpu.VMEM((_L * _D * 16,), jnp.float32)]
            + [pltpu.VMEM((_BW, _D), jnp.float32) for _ in range(2)]
            + [pltpu.VMEM((_D, _BW), jnp.float32) for _ in range(2)]
            + [pltpu.SemaphoreType.DMA for _ in range(4)]
        ),
    )
    def run(t_hbm, idx_hbm, pe_hbm, out_hbm, idx_v, pe_v,
            g0, g1, s0, s1, gs0, gs1, ss0, ss1):
        wid = lax.axis_index("s") * _NC + lax.axis_index("c")
        b0 = wid * _BW
        gbufs = (g0, g1)
        sbufs = (s0, s1)
        gsems = (gs0, gs1)
        ssems = (ss0, ss1)
        pltpu.sync_copy(idx_hbm.at[:, pl.ds(b0, _BW)], idx_v)
        pltpu.sync_copy(pe_hbm, pe_v)
        tok_iota = lax.iota(jnp.int32, 16)

        for b in range(2):
            pltpu.async_copy(t_hbm.at[idx_v.at[b]], gbufs[b], gsems[b])

        def pos_loop(i, carry):
            for b in range(2):
                l = i * 2 + b

                @pl.when(l < _L)
                def _do():
                    pltpu.make_async_copy(t_hbm.at[idx_v.at[l]], gbufs[b],
                                          gsems[b]).wait()
                    @pl.when(l >= 2)
                    def _ws():
                        pltpu.make_async_copy(
                            sbufs[b], out_hbm.at[l - 2, :, pl.ds(b0, _BW)],
                            ssems[b]).wait()

                    pe_base = l * (_D * 16)
                    for d in range(_D):
                        pe_vec = pe_v[pl.ds(pe_base + d * 16, 16)]
                        dcols = jnp.full((16,), d, jnp.int32)
                        for j in range(_BW // 16):
                            rows = tok_iota + (j * 16)
                            vals = plsc.load_gather(gbufs[b], (rows, dcols))
                            sbufs[b][d, pl.ds(j * 16, 16)] = vals + pe_vec
                    pltpu.async_copy(sbufs[b],
                                     out_hbm.at[l, :, pl.ds(b0, _BW)],
                                     ssems[b])
                    @pl.when(l + 2 < _L)
                    def _pf():
                        pltpu.async_copy(t_hbm.at[idx_v.at[l + 2]], gbufs[b],
                                         gsems[b])
            return carry

        lax.fori_loop(0, _L // 2, pos_loop, 0)

        for b in range(2):
            last = _L - 2 + b
            pltpu.make_async_copy(sbufs[b],
                                  out_hbm.at[last, :, pl.ds(b0, _BW)],
                                  ssems[b]).wait()

    return run(table_rm, idxT, peb)


def kernel(indices, table):
    idxT = jnp.transpose(indices.astype(jnp.int32))
    tT = jnp.transpose(table)
    peb = jnp.asarray(_pe_flat())

    table_rm = _relayout(tT)
    outT = _gather_pe(table_rm, idxT, peb)
    return jnp.transpose(outT, (2, 0, 1))


# speedup vs baseline: 1.1391x; 1.1391x over previous
import functools

import numpy as np
import jax
import jax.numpy as jnp
from jax import lax
from jax.experimental import pallas as pl
from jax.experimental.pallas import tpu as pltpu
from jax.experimental.pallas import tpu_sc as plsc

_VOCAB = 1000000
_D = 32
_B = 4096
_L = 50
_NC = 2
_NS = 16
_NW = _NC * _NS
_BW = _B // _NW

_CH = 512
_CHUNKS = 1952
_PER_W = _CHUNKS // _NW
_TAIL = _VOCAB - _CHUNKS * _CH
_TAIL_OFF = _CHUNKS * _CH
_TP = _D + 1
_SP = _BW + 1


def _pe_const() -> np.ndarray:
    pos = np.arange(_L, dtype=np.float32)[:, None]
    div = np.exp(np.arange(0, _D, 2, dtype=np.float32) * (-np.log(10000.0) / _D))
    pe = np.zeros((_L, _D), np.float32)
    pe[:, 0::2] = np.sin(pos * div)
    pe[:, 1::2] = np.cos(pos * div)
    return pe


def _make_mesh():
    return plsc.VectorSubcoreMesh(core_axis_name="c", subcore_axis_name="s")


def _relayout(tT):
    @functools.partial(
        pl.kernel,
        mesh=_make_mesh(),
        compiler_params=pltpu.CompilerParams(use_tc_tiling_on_sc=False,
                                             needs_layout_passes=False),
        out_type=jax.ShapeDtypeStruct((_VOCAB, _D), jnp.float32),
        scratch_types=(
            [pltpu.VMEM((_D, _CH), jnp.float32) for _ in range(2)]
            + [pltpu.VMEM((_CH, _TP), jnp.float32) for _ in range(2)]
            + [pltpu.VMEM((_D, _TAIL), jnp.float32),
               pltpu.VMEM((_TAIL, _TP), jnp.float32)]
            + [pltpu.SemaphoreType.DMA for _ in range(5)]
        ),
    )
    def run(tT_hbm, out_hbm, in0, in1, tr0, tr1, tin, ttr,
            gs0, gs1, ss0, ss1, tsem):
        wid = lax.axis_index("s") * _NC + lax.axis_index("c")
        ins = (in0, in1)
        trs = (tr0, tr1)
        gsems = (gs0, gs1)
        ssems = (ss0, ss1)
        lane = lax.iota(jnp.int32, 16)

        def col_of(i):
            return (wid + i * _NW) * _CH

        for b in range(2):
            pltpu.async_copy(tT_hbm.at[:, pl.ds(col_of(b), _CH)],
                             ins[b], gsems[b])

        def transpose_chunk(src, dst, ncols):
            def body(cg, carry):
                c0 = cg * 16
                rows = lane + c0
                for d in range(_D):
                    vals = src[d, pl.ds(c0, 16)]
                    dcol = jnp.full((16,), d, jnp.int32)
                    plsc.store_scatter(dst, (rows, dcol), vals)
                return carry
            lax.fori_loop(0, ncols // 16, body, 0)

        def chunk_loop(i, carry):
            for b in range(2):
                j = i * 2 + b

                @pl.when(j < _PER_W)
                def _do():
                    pltpu.make_async_copy(
                        tT_hbm.at[:, pl.ds(col_of(j), _CH)], ins[b],
                        gsems[b]).wait()

                    @pl.when(j >= 2)
                    def _ws():
                        pltpu.make_async_copy(
                            trs[b].at[:, pl.ds(0, _D)],
                            out_hbm.at[pl.ds(col_of(j - 2), _CH)],
                            ssems[b]).wait()

                    transpose_chunk(ins[b], trs[b], _CH)
                    pltpu.async_copy(trs[b].at[:, pl.ds(0, _D)],
                                     out_hbm.at[pl.ds(col_of(j), _CH)],
                                     ssems[b])

                    @pl.when(j + 2 < _PER_W)
                    def _pf():
                        pltpu.async_copy(
                            tT_hbm.at[:, pl.ds(col_of(j + 2), _CH)],
                            ins[b], gsems[b])
            return carry

        lax.fori_loop(0, (_PER_W + 1) // 2, chunk_loop, 0)

        for b in range(2):
            last = _PER_W - 1 - b
            pltpu.make_async_copy(trs[b].at[:, pl.ds(0, _D)],
                                  out_hbm.at[pl.ds(col_of(last), _CH)],
                                  ssems[b]).wait()

        @pl.when(wid == 0)
        def _tail():
            pltpu.async_copy(tT_hbm.at[:, pl.ds(_TAIL_OFF, _TAIL)], tin, tsem)
            pltpu.make_async_copy(tT_hbm.at[:, pl.ds(_TAIL_OFF, _TAIL)], tin,
                                  tsem).wait()
            transpose_chunk(tin, ttr, _TAIL)
            pltpu.async_copy(ttr.at[:, pl.ds(0, _D)],
                             out_hbm.at[pl.ds(_TAIL_OFF, _TAIL)], tsem)
            pltpu.make_async_copy(ttr.at[:, pl.ds(0, _D)],
                                  out_hbm.at[pl.ds(_TAIL_OFF, _TAIL)],
                                  tsem).wait()

    return run(tT)


def _gather_pe(table_rm, idxT, pe):
    @functools.partial(
        pl.kernel,
        mesh=_make_mesh(),
        compiler_params=pltpu.CompilerParams(use_tc_tiling_on_sc=False,
                                             needs_layout_passes=False),
        out_type=jax.ShapeDtypeStruct((_L, _D, _B), jnp.float32),
        scratch_types=(
            [pltpu.VMEM((_L, _BW), jnp.int32),
             pltpu.VMEM((_L, _D), jnp.float32)]
            + [pltpu.VMEM((_BW, _D), jnp.float32) for _ in range(2)]
            + [pltpu.VMEM((_D, _SP), jnp.float32) for _ in range(2)]
            + [pltpu.SemaphoreType.DMA for _ in range(4)]
        ),
    )
    def run(t_hbm, idx_hbm, pe_hbm, out_hbm, idx_v, pe_v,
            g0, g1, s0, s1, gs0, gs1, ss0, ss1):
        wid = lax.axis_index("s") * _NC + lax.axis_index("c")
        b0 = wid * _BW
        gbufs = (g0, g1)
        sbufs = (s0, s1)
        gsems = (gs0, gs1)
        ssems = (ss0, ss1)
        pltpu.sync_copy(idx_hbm.at[:, pl.ds(b0, _BW)], idx_v)
        pltpu.sync_copy(pe_hbm, pe_v)
        lane = lax.iota(jnp.int32, 16)
        drows = (lane, lane + 16)

        for b in range(2):
            pltpu.async_copy(t_hbm.at[idx_v.at[b]], gbufs[b], gsems[b])

        def pos_loop(i, carry):
            for b in range(2):
                l = i * 2 + b

                @pl.when(l < _L)
                def _do():
                    pltpu.make_async_copy(t_hbm.at[idx_v.at[l]], gbufs[b],
                                          gsems[b]).wait()

                    @pl.when(l >= 2)
                    def _ws():
                        pltpu.make_async_copy(
                            sbufs[b].at[:, pl.ds(0, _BW)],
                            out_hbm.at[l - 2, :, pl.ds(b0, _BW)],
                            ssems[b]).wait()

                    pe_lo = pe_v[l, pl.ds(0, 16)]
                    pe_hi = pe_v[l, pl.ds(16, 16)]
                    pes = (pe_lo, pe_hi)

                    def tok_body(t8, c2):
                        for u in range(8):
                            t = t8 * 8 + u
                            tcol = jnp.full((16,), t, jnp.int32)
                            for h in range(2):
                                vals = gbufs[b][t, pl.ds(16 * h, 16)] + pes[h]
                                plsc.store_scatter(sbufs[b],
                                                   (drows[h], tcol), vals)
                        return c2
                    lax.fori_loop(0, _BW // 8, tok_body, 0)

                    pltpu.async_copy(sbufs[b].at[:, pl.ds(0, _BW)],
                                     out_hbm.at[l, :, pl.ds(b0, _BW)],
                                     ssems[b])

                    @pl.when(l + 2 < _L)
                    def _pf():
                        pltpu.async_copy(t_hbm.at[idx_v.at[l + 2]], gbufs[b],
                                         gsems[b])
            return carry

        lax.fori_loop(0, _L // 2, pos_loop, 0)

        for b in range(2):
            last = _L - 2 + b
            pltpu.make_async_copy(sbufs[b].at[:, pl.ds(0, _BW)],
                                  out_hbm.at[last, :, pl.ds(b0, _BW)],
                                  ssems[b]).wait()

    return run(table_rm, idxT, pe)


def kernel(indices, table):
    idxT = jnp.transpose(indices.astype(jnp.int32))
    tT = jnp.transpose(table)
    pe = jnp.asarray(_pe_const())

    table_rm = _relayout(tT)
    outT = _gather_pe(table_rm, idxT, pe)
    return jnp.transpose(outT, (2, 0, 1))


# speedup vs baseline: 5.1404x; 4.5126x over previous
import functools

import numpy as np
import jax
import jax.numpy as jnp
from jax import lax
from jax.experimental import pallas as pl
from jax.experimental.pallas import tpu as pltpu
from jax.experimental.pallas import tpu_sc as plsc

_VOCAB = 1000000
_D = 32
_B = 4096
_L = 50
_NC = 2
_NS = 16
_NW = _NC * _NS
_SEQ_W = _B // _NW
_NB = 8
_ROUNDS = _SEQ_W // _NB


def _pe_const() -> np.ndarray:
    pos = np.arange(_L, dtype=np.float32)[:, None]
    div = np.exp(np.arange(0, _D, 2, dtype=np.float32) * (-np.log(10000.0) / _D))
    pe = np.zeros((_L, _D), np.float32)
    pe[:, 0::2] = np.sin(pos * div)
    pe[:, 1::2] = np.cos(pos * div)
    return pe


def kernel(indices, table):
    idx = indices.astype(jnp.int32)
    pe = jnp.asarray(_pe_const())

    mesh = plsc.VectorSubcoreMesh(core_axis_name="c", subcore_axis_name="s")

    @functools.partial(
        pl.kernel,
        mesh=mesh,
        compiler_params=pltpu.CompilerParams(use_tc_tiling_on_sc=False),
        out_type=jax.ShapeDtypeStruct((_B, _L, _D), jnp.float32),
        scratch_types=(
            [pltpu.VMEM((_SEQ_W, _L), jnp.int32),
             pltpu.VMEM((_L, _D), jnp.float32)]
            + [pltpu.VMEM((_L, _D), jnp.float32) for _ in range(2 * _NB)]
            + [pltpu.SemaphoreType.DMA for _ in range(2 * _NB)]
        ),
    )
    def run(table_hbm, idx_hbm, pe_hbm, out_hbm, idx_v, pe_v, *bufs_sems):
        gbuf = bufs_sems[:_NB]
        sbuf = bufs_sems[_NB:2 * _NB]
        gsem = bufs_sems[2 * _NB:3 * _NB]
        ssem = bufs_sems[3 * _NB:]
        wid = lax.axis_index("s") * _NC + lax.axis_index("c")
        pltpu.sync_copy(idx_hbm.at[pl.ds(wid * _SEQ_W, _SEQ_W)], idx_v)
        pltpu.sync_copy(pe_hbm, pe_v)
        base = wid * _SEQ_W

        for b in range(_NB):
            pltpu.async_copy(table_hbm.at[idx_v.at[b]], gbuf[b], gsem[b])

        def round_body(t, carry):
            for b in range(_NB):
                j = t * _NB + b
                @pl.when(t > 0)
                def _wait_store():
                    pltpu.make_async_copy(sbuf[b], out_hbm.at[base + j],
                                          ssem[b]).wait()

                pltpu.make_async_copy(table_hbm.at[idx_v.at[j]], gbuf[b],
                                      gsem[b]).wait()
                for i in range(_L):
                    for h in range(_D // 16):
                        sl = pl.ds(h * 16, 16)
                        sbuf[b][i, sl] = gbuf[b][i, sl] + pe_v[i, sl]
                jn = lax.min(j + _NB, _SEQ_W - 1)
                pltpu.async_copy(table_hbm.at[idx_v.at[jn]], gbuf[b], gsem[b])
                pltpu.async_copy(sbuf[b], out_hbm.at[base + j], ssem[b])
            return carry

        lax.fori_loop(0, _ROUNDS, round_body, 0)

        for b in range(_NB):
            pltpu.make_async_copy(table_hbm.at[idx_v.at[0]], gbuf[b],
                                  gsem[b]).wait()
            pltpu.make_async_copy(sbuf[b], out_hbm.at[0], ssem[b]).wait()

    return run(table, idx, pe)
